# baseline (speedup 1.0000x reference)
import numpy as np
import jax
import jax.numpy as jnp
from jax.experimental import pallas as pl


def _build_perm_matrices():
    P = np.zeros((2, 256, 256), dtype=np.float32)
    for v in (0, 1):
        for hi in (0, 1):
            for l in range(128):
                o = hi ^ (l & 1)
                s = (l ^ (l >> 1)) ^ (64 * v)
                P[v, o * 128 + s, hi * 128 + l] = 1.0
    return P


_P = jnp.asarray(_build_perm_matrices())


def _body(p_ref, a_ref, b_ref, o_ref):
    a = a_ref[:, 0, 0, :]
    b = b_ref[:, 0, 0, :]
    p = p_ref[0]
    res = jax.lax.dot_general(
        a, p[:128], (((1,), (0,)), ((), ())),
        preferred_element_type=jnp.float32,
    ) + jax.lax.dot_general(
        b, p[128:], (((1,), (0,)), ((), ())),
        preferred_element_type=jnp.float32,
    )
    o_ref[:, :, 0, 0, :] = res.reshape(128, 2, 128)


def kernel(phi):
    phi4 = phi.reshape(128, 512, 1, 128)

    def blk(v, m):
        i = 2 * m + v
        return i ^ (i >> 1)

    out5 = pl.pallas_call(
        _body,
        grid=(2, 128),
        in_specs=[
            pl.BlockSpec((1, 256, 256), lambda v, m: (v, 0, 0)),
            pl.BlockSpec((128, 1, 1, 128), lambda v, m: (0, blk(v, m), 0, 0)),
            pl.BlockSpec(
                (128, 1, 1, 128), lambda v, m: (0, blk(v, m) ^ 384, 0, 0)
            ),
        ],
        out_specs=pl.BlockSpec(
            (128, 2, 1, 1, 128), lambda v, m: (0, 0, 2 * m + v, 0, 0)
        ),
        out_shape=jax.ShapeDtypeStruct((128, 2, 256, 1, 128), jnp.float32),
    )(_P, phi4, phi4)
    return out5.reshape(128, 65536)


# speedup vs baseline: 1.0990x; 1.0990x over previous
import numpy as np
import jax
import jax.numpy as jnp
from jax.experimental import pallas as pl

_G = 16
_H = _G // 2
_NGRP = 256 // _G


def _build_perm_matrices():
    P = np.zeros((2, 256, 256), dtype=np.float32)
    for v in (0, 1):
        for hi in (0, 1):
            for l in range(128):
                o = hi ^ (l & 1)
                s = (l ^ (l >> 1)) ^ (64 * v)
                P[v, o * 128 + s, hi * 128 + l] = 1.0
    return P


_P_NP = _build_perm_matrices()


def _body(p0_ref, p1_ref, a0_ref, a1_ref, b0_ref, b1_ref, o_ref):
    for t in range(_G):
        a_ref = a0_ref if (t >> 3) == 0 else a1_ref
        b_ref = b0_ref if (t >> 3) == 0 else b1_ref
        r = (t ^ (t >> 1)) & (_H - 1)
        sl = slice(r * 128, (r + 1) * 128)
        a = a_ref[:, 0, 0, sl]
        b = b_ref[:, 0, 0, sl]
        p = (p0_ref if t % 2 == 0 else p1_ref)[:, :]
        res = jax.lax.dot_general(
            a, p[:128], (((1,), (0,)), ((), ())),
            preferred_element_type=jnp.float32,
        ) + jax.lax.dot_general(
            b, p[128:], (((1,), (0,)), ((), ())),
            preferred_element_type=jnp.float32,
        )
        ot = slice(t * 128, (t + 1) * 128)
        o_ref[:, 0, 0, 0, ot] = res[:, :128]
        o_ref[:, 1, 0, 0, ot] = res[:, 128:]


def kernel(phi):
    phiH = phi.reshape(128, 512 // _H, 1, _H * 128)
    p_mat = jnp.asarray(_P_NP)

    def ga(v, m):
        i = _G * (2 * m + v)
        return (i ^ (i >> 1)) >> 3

    out5 = pl.pallas_call(
        _body,
        grid=(2, _NGRP // 2),
        in_specs=[
            pl.BlockSpec((256, 256), lambda v, m: (0, 0)),
            pl.BlockSpec((256, 256), lambda v, m: (0, 0)),
            pl.BlockSpec((128, 1, 1, _H * 128),
                         lambda v, m: (0, ga(v, m), 0, 0)),
            pl.BlockSpec((128, 1, 1, _H * 128),
                         lambda v, m: (0, ga(v, m) ^ 1, 0, 0)),
            pl.BlockSpec((128, 1, 1, _H * 128),
                         lambda v, m: (0, ga(v, m) ^ 48, 0, 0)),
            pl.BlockSpec((128, 1, 1, _H * 128),
                         lambda v, m: (0, ga(v, m) ^ 48 ^ 1, 0, 0)),
        ],
        out_specs=pl.BlockSpec(
            (128, 2, 1, 1, _G * 128),
            lambda v, m: (0, 0, 2 * m + v, 0, 0),
        ),
        out_shape=jax.ShapeDtypeStruct(
            (128, 2, _NGRP, 1, _G * 128), jnp.float32
        ),
    )(p_mat[0], p_mat[1], phiH, phiH, phiH, phiH)
    return out5.reshape(128, 65536)


# speedup vs baseline: 6.5648x; 5.9733x over previous
import numpy as np
import jax
import jax.numpy as jnp
from jax.experimental import pallas as pl

_G = 16
_H = _G // 2
_NGRP = 512 // _G


def _build_perm_matrices():
    P = np.zeros((2, 256, 128), dtype=np.float32)
    for v in (0, 1):
        for l in range(128):
            o = l & 1
            s = (l ^ (l >> 1)) ^ (64 * v)
            P[v, o * 128 + s, l] = 1.0
    return P


_P_NP = _build_perm_matrices()


def _body(pe_ref, po_ref, a0_ref, a1_ref, b0_ref, b1_ref, o_ref):
    pe = pe_ref[:, :]
    po = po_ref[:, :]
    pparts = ((pe[:128], pe[128:]), (po[:128], po[128:]))
    for t in range(_G):
        a_ref = a0_ref if (t >> 3) == 0 else a1_ref
        b_ref = b0_ref if (t >> 3) == 0 else b1_ref
        r = (t ^ (t >> 1)) & (_H - 1)
        sl = slice(r * 128, (r + 1) * 128)
        pa, pb = pparts[t & 1]
        res = jax.lax.dot_general(
            a_ref[:, sl], pa, (((1,), (0,)), ((), ())),
            preferred_element_type=jnp.float32,
        ) + jax.lax.dot_general(
            b_ref[:, sl], pb, (((1,), (0,)), ((), ())),
            preferred_element_type=jnp.float32,
        )
        o_ref[:, t * 128:(t + 1) * 128] = res


def kernel(phi):
    p_mat = jnp.asarray(_P_NP)

    def ha(v, m):
        i = _G * (2 * m + v)
        return (i ^ (i >> 1)) >> 3

    out = pl.pallas_call(
        _body,
        grid=(2, _NGRP // 2),
        in_specs=[
            pl.BlockSpec((256, 128), lambda v, m: (0, 0)),
            pl.BlockSpec((256, 128), lambda v, m: (0, 0)),
            pl.BlockSpec((128, _H * 128), lambda v, m: (0, ha(v, m))),
            pl.BlockSpec((128, _H * 128), lambda v, m: (0, ha(v, m) ^ 1)),
            pl.BlockSpec((128, _H * 128), lambda v, m: (0, ha(v, m) ^ 48)),
            pl.BlockSpec((128, _H * 128), lambda v, m: (0, ha(v, m) ^ 49)),
        ],
        out_specs=pl.BlockSpec(
            (128, _G * 128), lambda v, m: (0, 2 * m + v)
        ),
        out_shape=jax.ShapeDtypeStruct((128, 65536), jnp.float32),
    )(p_mat[0], p_mat[1], phi, phi, phi, phi)
    return out
